# baseline (speedup 1.0000x reference)
import functools

import jax
import jax.numpy as jnp
from jax import lax
from jax.experimental import pallas as pl
from jax.experimental.pallas import tpu as pltpu
from jax.experimental.pallas import tpu_sc as plsc

_N = 10000
_E = 320000
_F = 128
_NC = 2
_NS = 16
_NW = _NC * _NS
_CB = 128
_EPW = 10240
_EP = _NW * _EPW
_NCH = _EPW // _CB
_IBC = 16
_NIB = _NCH // _IBC
_NPD = 10240
_RPD = _NPD // _NS
_NPP = 10112
_RPP = _NPP // _NS
_DUMMY = _NPP - 1

_mesh = plsc.VectorSubcoreMesh(core_axis_name="c", subcore_axis_name="s")


@functools.partial(
    pl.kernel,
    out_type=jax.ShapeDtypeStruct((_NC, _NPD), jnp.float32),
    mesh=_mesh,
    scratch_types=[
        pltpu.VMEM((_NCH, _CB), jnp.int32),
        pltpu.VMEM((_CB,), jnp.float32),
        pltpu.SemaphoreType.DMA,
        pltpu.VMEM_SHARED((_NPD,), jnp.float32),
    ],
)
def _deg_kernel(dst_hbm, zeros_hbm, out_hbm, dst_v, ones_v, sem, acc):
    c = lax.axis_index("c")
    s = lax.axis_index("s")
    w = s * _NC + c
    pltpu.sync_copy(dst_hbm.at[w], dst_v)
    for i in range(_CB // 16):
        ones_v[pl.ds(16 * i, 16)] = jnp.full((16,), 1.0, jnp.float32)
    pltpu.sync_copy(zeros_hbm.at[pl.ds(s * _RPD, _RPD)],
                    acc.at[pl.ds(s * _RPD, _RPD)])
    plsc.subcore_barrier()

    def fire(j, carry):
        pltpu.async_copy(ones_v, acc.at[dst_v.at[j]], sem, add=True)
        return carry

    lax.fori_loop(0, _NCH, fire, 0)

    def drain(j, carry):
        pltpu.make_async_copy(ones_v, acc.at[dst_v.at[0]], sem).wait()
        return carry

    lax.fori_loop(0, _NCH, drain, 0)
    plsc.subcore_barrier()
    pltpu.sync_copy(acc.at[pl.ds(s * _RPD, _RPD)],
                    out_hbm.at[c, pl.ds(s * _RPD, _RPD)])


@functools.partial(
    pl.kernel,
    out_type=jax.ShapeDtypeStruct((_NC, _NPP, _F), jnp.float32),
    mesh=_mesh,
    scratch_types=[
        pltpu.VMEM((_IBC, _CB), jnp.int32),
        pltpu.VMEM((_IBC, _CB), jnp.int32),
        pltpu.VMEM((2, _CB, _F), jnp.float32),
        pltpu.SemaphoreType.DMA,
        pltpu.SemaphoreType.DMA,
        pltpu.VMEM_SHARED((_NPP, _F), jnp.float32),
    ],
)
def _prop_kernel(xs_hbm, src_hbm, dst_hbm, zeros_hbm, out_hbm,
                 src_v, dst_v, buf, sem0, sem1, acc):
    c = lax.axis_index("c")
    s = lax.axis_index("s")
    w = s * _NC + c
    pltpu.sync_copy(zeros_hbm.at[pl.ds(s * _RPP, _RPP)],
                    acc.at[pl.ds(s * _RPP, _RPP)])
    plsc.subcore_barrier()

    sems = (sem0, sem1)

    def gissue(jj, b):
        pltpu.async_copy(xs_hbm.at[src_v.at[jj]], buf.at[b], sems[b])

    def gwait(jj, b):
        pltpu.make_async_copy(xs_hbm.at[src_v.at[jj]], buf.at[b],
                              sems[b]).wait()

    def scat(jj, b):
        pltpu.sync_copy(buf.at[b], acc.at[dst_v.at[jj]], add=True)

    def blk_body(blk, carry):
        pltpu.sync_copy(src_hbm.at[w * _NIB + blk], src_v)
        pltpu.sync_copy(dst_hbm.at[w * _NIB + blk], dst_v)
        gissue(0, 0)
        gissue(1, 1)

        def pair(j2, inner):
            j = 2 * j2
            gwait(j, 0)
            scat(j, 0)
            gissue(j + 2, 0)
            gwait(j + 1, 1)
            scat(j + 1, 1)
            gissue(j + 3, 1)
            return inner

        lax.fori_loop(0, _IBC // 2 - 1, pair, 0)
        gwait(_IBC - 2, 0)
        scat(_IBC - 2, 0)
        gwait(_IBC - 1, 1)
        scat(_IBC - 1, 1)
        return carry

    lax.fori_loop(0, _NIB, blk_body, 0)
    plsc.subcore_barrier()
    pltpu.sync_copy(acc.at[pl.ds(s * _RPP, _RPP)],
                    out_hbm.at[c, pl.ds(s * _RPP, _RPP)])


def _tc_first(x_ref, w_ref, deg_ref, xs_ref, dinv_ref):
    dinv = lax.rsqrt(deg_ref[...])
    h = jnp.dot(x_ref[...], w_ref[...], preferred_element_type=jnp.float32)
    dinv_ref[...] = dinv
    xs_ref[...] = dinv * h


def _tc_mid(za_ref, zb_ref, xs_ref, dinv_ref, b_ref, w_ref, out_ref):
    z = za_ref[...] + zb_ref[...] + xs_ref[...]
    x1 = jnp.maximum(dinv_ref[...] * z + b_ref[...], 0.0)
    h = jnp.dot(x1, w_ref[...], preferred_element_type=jnp.float32)
    out_ref[...] = dinv_ref[...] * h


def _tc_last(za_ref, zb_ref, xs_ref, dinv_ref, b_ref, wl_ref, bl_ref, out_ref):
    z = za_ref[...] + zb_ref[...] + xs_ref[...]
    x2 = jnp.maximum(dinv_ref[...] * z + b_ref[...], 0.0)
    out_ref[...] = (jnp.dot(x2, wl_ref[...], preferred_element_type=jnp.float32)
                    + bl_ref[...])


def kernel(node_features, edge_indices, W1, b1, W2, b2, Wl, bl):
    ei = edge_indices.astype(jnp.int32)
    pad = _EP - _E
    src = jnp.concatenate([ei[0], jnp.zeros((pad,), jnp.int32)])
    dst = jnp.concatenate([ei[1], jnp.full((pad,), _DUMMY, jnp.int32)])
    src_blk = src.reshape(_NW * _NIB, _IBC, _CB)
    dst_blk = dst.reshape(_NW * _NIB, _IBC, _CB)
    dst_w = dst.reshape(_NW, _NCH, _CB)
    zeros_f = jnp.zeros((_NPP, _F), jnp.float32)
    zeros_1 = jnp.zeros((_NPD,), jnp.float32)

    degp = _deg_kernel(dst_w, zeros_1)
    deg_col = (degp[0, :_N] + degp[1, :_N] + 1.0)[:, None]

    xs1, dinv = pl.pallas_call(
        _tc_first,
        out_shape=[
            jax.ShapeDtypeStruct((_N, _F), jnp.float32),
            jax.ShapeDtypeStruct((_N, 1), jnp.float32),
        ],
    )(node_features, W1, deg_col)

    z1 = _prop_kernel(xs1, src_blk, dst_blk, zeros_f)

    xs2 = pl.pallas_call(
        _tc_mid,
        out_shape=jax.ShapeDtypeStruct((_N, _F), jnp.float32),
    )(z1[0, :_N], z1[1, :_N], xs1, dinv, b1.reshape(1, _F), W2)

    z2 = _prop_kernel(xs2, src_blk, dst_blk, zeros_f)

    out = pl.pallas_call(
        _tc_last,
        out_shape=jax.ShapeDtypeStruct((_N, 40), jnp.float32),
    )(z2[0, :_N], z2[1, :_N], xs2, dinv, b2.reshape(1, _F), Wl,
      bl.reshape(1, 40))
    return out


# speedup vs baseline: 1.2846x; 1.2846x over previous
import functools

import jax
import jax.numpy as jnp
from jax import lax
from jax.experimental import pallas as pl
from jax.experimental.pallas import tpu as pltpu
from jax.experimental.pallas import tpu_sc as plsc

_N = 10000
_E = 320000
_F = 128
_NC = 2
_NS = 16
_NW = _NC * _NS
_CB = 128
_EPW = 10240
_EP = _NW * _EPW
_NCH = _EPW // _CB
_IBC = 16
_NIB = _NCH // _IBC
_NPD = 10240
_RPD = _NPD // _NS
_NPP = 10112
_RPP = _NPP // _NS
_DUMMY = _NPP - 1

_mesh = plsc.VectorSubcoreMesh(core_axis_name="c", subcore_axis_name="s")


@functools.partial(
    pl.kernel,
    out_type=jax.ShapeDtypeStruct((_NC, _NPD), jnp.float32),
    mesh=_mesh,
    scratch_types=[
        pltpu.VMEM((_NCH, _CB), jnp.int32),
        pltpu.VMEM((_CB,), jnp.float32),
        pltpu.SemaphoreType.DMA,
        pltpu.VMEM_SHARED((_NPD,), jnp.float32),
    ],
)
def _deg_kernel(dst_hbm, zeros_hbm, out_hbm, dst_v, ones_v, sem, acc):
    c = lax.axis_index("c")
    s = lax.axis_index("s")
    w = s * _NC + c
    pltpu.sync_copy(dst_hbm.at[w], dst_v)
    for i in range(_CB // 16):
        ones_v[pl.ds(16 * i, 16)] = jnp.full((16,), 1.0, jnp.float32)
    pltpu.sync_copy(zeros_hbm.at[pl.ds(s * _RPD, _RPD)],
                    acc.at[pl.ds(s * _RPD, _RPD)])
    plsc.subcore_barrier()

    def fire(j, carry):
        pltpu.async_copy(ones_v, acc.at[dst_v.at[j]], sem, add=True)
        return carry

    lax.fori_loop(0, _NCH, fire, 0)

    def drain(j, carry):
        pltpu.make_async_copy(ones_v, acc.at[dst_v.at[0]], sem).wait()
        return carry

    lax.fori_loop(0, _NCH, drain, 0)
    plsc.subcore_barrier()
    pltpu.sync_copy(acc.at[pl.ds(s * _RPD, _RPD)],
                    out_hbm.at[c, pl.ds(s * _RPD, _RPD)])


@functools.partial(
    pl.kernel,
    out_type=jax.ShapeDtypeStruct((_NC, _NPP, _F), jnp.float32),
    mesh=_mesh,
    scratch_types=[
        pltpu.VMEM((_IBC, _CB), jnp.int32),
        pltpu.VMEM((_IBC, _CB), jnp.int32),
        pltpu.VMEM((2, _CB, _F), jnp.float32),
        pltpu.SemaphoreType.DMA,
        pltpu.SemaphoreType.DMA,
        pltpu.VMEM_SHARED((_NPP, _F), jnp.float32),
    ],
)
def _prop_kernel(xs_hbm, src_hbm, dst_hbm, zeros_hbm, out_hbm,
                 src_v, dst_v, buf, sem0, sem1, acc):
    c = lax.axis_index("c")
    s = lax.axis_index("s")
    w = s * _NC + c
    pltpu.sync_copy(zeros_hbm.at[pl.ds(s * _RPP, _RPP)],
                    acc.at[pl.ds(s * _RPP, _RPP)])
    plsc.subcore_barrier()

    sems = (sem0, sem1)

    def gissue(jj, b):
        pltpu.async_copy(xs_hbm.at[src_v.at[jj]], buf.at[b], sems[b])

    def gwait(jj, b):
        pltpu.make_async_copy(xs_hbm.at[src_v.at[jj]], buf.at[b],
                              sems[b]).wait()

    def scat(jj, b):
        pltpu.sync_copy(buf.at[b], acc.at[dst_v.at[jj]], add=True)

    def blk_body(blk, carry):
        pltpu.sync_copy(src_hbm.at[w * _NIB + blk], src_v)
        pltpu.sync_copy(dst_hbm.at[w * _NIB + blk], dst_v)
        gissue(0, 0)
        gissue(1, 1)

        def pair(j2, inner):
            j = 2 * j2
            gwait(j, 0)
            scat(j, 0)
            gissue(j + 2, 0)
            gwait(j + 1, 1)
            scat(j + 1, 1)
            gissue(j + 3, 1)
            return inner

        lax.fori_loop(0, _IBC // 2 - 1, pair, 0)
        gwait(_IBC - 2, 0)
        scat(_IBC - 2, 0)
        gwait(_IBC - 1, 1)
        scat(_IBC - 1, 1)
        return carry

    lax.fori_loop(0, _NIB, blk_body, 0)
    plsc.subcore_barrier()
    pltpu.sync_copy(acc.at[pl.ds(s * _RPP, _RPP)],
                    out_hbm.at[c, pl.ds(s * _RPP, _RPP)])


def _tc_first(x_ref, w_ref, deg_ref, xs_ref, dinv_ref):
    dinv = lax.rsqrt(deg_ref[...])
    h = jnp.dot(x_ref[...], w_ref[...], preferred_element_type=jnp.float32)
    dinv_ref[...] = dinv
    xs_ref[...] = dinv * h


def _tc_mid(za_ref, zb_ref, xs_ref, dinv_ref, b_ref, w_ref, out_ref):
    z = za_ref[...] + zb_ref[...] + xs_ref[...]
    x1 = jnp.maximum(dinv_ref[...] * z + b_ref[...], 0.0)
    h = jnp.dot(x1, w_ref[...], preferred_element_type=jnp.float32)
    out_ref[...] = dinv_ref[...] * h


def _tc_last(za_ref, zb_ref, xs_ref, dinv_ref, b_ref, wl_ref, bl_ref, out_ref):
    z = za_ref[...] + zb_ref[...] + xs_ref[...]
    x2 = jnp.maximum(dinv_ref[...] * z + b_ref[...], 0.0)
    out_ref[...] = (jnp.dot(x2, wl_ref[...], preferred_element_type=jnp.float32)
                    + bl_ref[...])


def kernel(node_features, edge_indices, W1, b1, W2, b2, Wl, bl):
    ei = edge_indices.astype(jnp.int32)
    epw_real = _E // _NW
    padw = _EPW - epw_real
    dmy = _N + (jnp.arange(padw, dtype=jnp.int32) % (_NPP - _N))
    src = jnp.concatenate(
        [ei[0].reshape(_NW, epw_real),
         jnp.zeros((_NW, padw), jnp.int32)], axis=1)
    dst = jnp.concatenate(
        [ei[1].reshape(_NW, epw_real),
         jnp.broadcast_to(dmy, (_NW, padw))], axis=1)
    src_blk = src.reshape(_NW * _NIB, _IBC, _CB)
    dst_blk = dst.reshape(_NW * _NIB, _IBC, _CB)
    dst_w = dst.reshape(_NW, _NCH, _CB)
    zeros_f = jnp.zeros((_NPP, _F), jnp.float32)
    zeros_1 = jnp.zeros((_NPD,), jnp.float32)

    degp = _deg_kernel(dst_w, zeros_1)
    deg_col = (degp[0, :_N] + degp[1, :_N] + 1.0)[:, None]

    xs1, dinv = pl.pallas_call(
        _tc_first,
        out_shape=[
            jax.ShapeDtypeStruct((_N, _F), jnp.float32),
            jax.ShapeDtypeStruct((_N, 1), jnp.float32),
        ],
    )(node_features, W1, deg_col)

    z1 = _prop_kernel(xs1, src_blk, dst_blk, zeros_f)

    xs2 = pl.pallas_call(
        _tc_mid,
        out_shape=jax.ShapeDtypeStruct((_N, _F), jnp.float32),
    )(z1[0, :_N], z1[1, :_N], xs1, dinv, b1.reshape(1, _F), W2)

    z2 = _prop_kernel(xs2, src_blk, dst_blk, zeros_f)

    out = pl.pallas_call(
        _tc_last,
        out_shape=jax.ShapeDtypeStruct((_N, 40), jnp.float32),
    )(z2[0, :_N], z2[1, :_N], xs2, dinv, b2.reshape(1, _F), Wl,
      bl.reshape(1, 40))
    return out


# speedup vs baseline: 3.4470x; 2.6833x over previous
import functools

import jax
import jax.numpy as jnp
from jax import lax
from jax.experimental import pallas as pl
from jax.experimental.pallas import tpu as pltpu
from jax.experimental.pallas import tpu_sc as plsc

_N = 10000
_E = 320000
_F = 128
_NC = 2
_NS = 16
_NW = _NC * _NS
_EPW = _E // _NW
_CB = 80
_NCH = _EPW // _CB
_IBC = 25
_NIB = _NCH // _IBC
_NP = 10240
_RPT = _NP // _NS

_mesh = plsc.VectorSubcoreMesh(core_axis_name="c", subcore_axis_name="s")


@functools.partial(
    pl.kernel,
    out_type=jax.ShapeDtypeStruct((_NC, _NP), jnp.float32),
    mesh=_mesh,
    scratch_types=[
        pltpu.VMEM((_NCH, _CB), jnp.int32),
        pltpu.VMEM((_CB,), jnp.float32),
        pltpu.SemaphoreType.DMA,
        pltpu.VMEM_SHARED((_NP,), jnp.float32),
    ],
)
def _deg_kernel(dst_hbm, zeros_hbm, out_hbm, dst_v, ones_v, sem, acc):
    c = lax.axis_index("c")
    s = lax.axis_index("s")
    w = s * _NC + c
    for blk in range(_NIB):
        pltpu.sync_copy(dst_hbm.at[w * _NIB + blk],
                        dst_v.at[pl.ds(blk * _IBC, _IBC)])
    for i in range(_CB // 16):
        ones_v[pl.ds(16 * i, 16)] = jnp.full((16,), 1.0, jnp.float32)
    pltpu.sync_copy(zeros_hbm.at[pl.ds(s * _RPT, _RPT)],
                    acc.at[pl.ds(s * _RPT, _RPT)])
    plsc.subcore_barrier()

    def fire(j, carry):
        pltpu.async_copy(ones_v, acc.at[dst_v.at[j]], sem, add=True)
        return carry

    lax.fori_loop(0, _NCH, fire, 0)

    def drain(j, carry):
        pltpu.make_async_copy(ones_v, acc.at[dst_v.at[0]], sem).wait()
        return carry

    lax.fori_loop(0, _NCH, drain, 0)
    plsc.subcore_barrier()
    pltpu.sync_copy(acc.at[pl.ds(s * _RPT, _RPT)],
                    out_hbm.at[c, pl.ds(s * _RPT, _RPT)])


@functools.partial(
    pl.kernel,
    out_type=jax.ShapeDtypeStruct((_NC, _NP, _F), jnp.float32),
    mesh=_mesh,
    scratch_types=[
        pltpu.VMEM((_IBC, _CB), jnp.int32),
        pltpu.VMEM((_IBC, _CB), jnp.int32),
        pltpu.VMEM((2, _CB, _F), jnp.float32),
        pltpu.SemaphoreType.DMA,
        pltpu.SemaphoreType.DMA,
        pltpu.SemaphoreType.DMA,
        pltpu.VMEM_SHARED((_NP, _F), jnp.float32),
    ],
)
def _prop_kernel(xs_hbm, src_hbm, dst_hbm, zeros_hbm, out_hbm,
                 src_v, dst_v, buf, sem0, sem1, zsem, acc):
    c = lax.axis_index("c")
    s = lax.axis_index("s")
    w = s * _NC + c

    sems = (sem0, sem1)

    def gissue(jj, b):
        pltpu.async_copy(xs_hbm.at[src_v.at[jj]], buf.at[b], sems[b])

    def gwait(jj, b):
        pltpu.make_async_copy(xs_hbm.at[src_v.at[jj]], buf.at[b],
                              sems[b]).wait()

    def scat(jj, b):
        pltpu.sync_copy(buf.at[b], acc.at[dst_v.at[jj]], add=True)

    def pipeline():
        def pair(j2, inner):
            j = 2 * j2
            gissue(j + 1, 1)
            gwait(j, 0)
            scat(j, 0)
            gissue(j + 2, 0)
            gwait(j + 1, 1)
            scat(j + 1, 1)
            return inner

        lax.fori_loop(0, (_IBC - 1) // 2, pair, 0)
        gwait(_IBC - 1, 0)
        scat(_IBC - 1, 0)

    pltpu.async_copy(zeros_hbm.at[pl.ds(s * _RPT, _RPT)],
                     acc.at[pl.ds(s * _RPT, _RPT)], zsem)
    pltpu.sync_copy(src_hbm.at[w * _NIB], src_v)
    pltpu.sync_copy(dst_hbm.at[w * _NIB], dst_v)
    gissue(0, 0)
    pltpu.make_async_copy(zeros_hbm.at[pl.ds(s * _RPT, _RPT)],
                          acc.at[pl.ds(s * _RPT, _RPT)], zsem).wait()
    plsc.subcore_barrier()
    pipeline()

    def blk_body(blk, carry):
        pltpu.sync_copy(src_hbm.at[w * _NIB + blk], src_v)
        pltpu.sync_copy(dst_hbm.at[w * _NIB + blk], dst_v)
        gissue(0, 0)
        pipeline()
        return carry

    lax.fori_loop(1, _NIB, blk_body, 0)
    plsc.subcore_barrier()
    pltpu.sync_copy(acc.at[pl.ds(s * _RPT, _RPT)],
                    out_hbm.at[c, pl.ds(s * _RPT, _RPT)])


def _tc_first(x_ref, w_ref, deg_ref, xs_ref, dinv_ref):
    dinv = lax.rsqrt(deg_ref[...])
    h = jnp.dot(x_ref[...], w_ref[...], preferred_element_type=jnp.float32)
    dinv_ref[...] = dinv
    xs_ref[...] = dinv * h


def _tc_mid(z_ref, xs_ref, dinv_ref, b_ref, w_ref, out_ref):
    z = z_ref[0, :_N, :] + z_ref[1, :_N, :] + xs_ref[...]
    x1 = jnp.maximum(dinv_ref[...] * z + b_ref[...], 0.0)
    h = jnp.dot(x1, w_ref[...], preferred_element_type=jnp.float32)
    out_ref[...] = dinv_ref[...] * h


def _tc_last(z_ref, xs_ref, dinv_ref, b_ref, wl_ref, bl_ref, out_ref):
    z = z_ref[0, :_N, :] + z_ref[1, :_N, :] + xs_ref[...]
    x2 = jnp.maximum(dinv_ref[...] * z + b_ref[...], 0.0)
    out_ref[...] = (jnp.dot(x2, wl_ref[...], preferred_element_type=jnp.float32)
                    + bl_ref[...])


def kernel(node_features, edge_indices, W1, b1, W2, b2, Wl, bl):
    ei = edge_indices.astype(jnp.int32)
    src_blk = ei[0].reshape(_NW * _NIB, _IBC, _CB)
    dst_blk = ei[1].reshape(_NW * _NIB, _IBC, _CB)
    zeros_f = jnp.zeros((_NP, _F), jnp.float32)
    zeros_1 = jnp.zeros((_NP,), jnp.float32)

    degp = _deg_kernel(dst_blk, zeros_1)
    deg_col = (degp[0, :_N] + degp[1, :_N] + 1.0)[:, None]

    xs1, dinv = pl.pallas_call(
        _tc_first,
        out_shape=[
            jax.ShapeDtypeStruct((_N, _F), jnp.float32),
            jax.ShapeDtypeStruct((_N, 1), jnp.float32),
        ],
    )(node_features, W1, deg_col)

    z1 = _prop_kernel(xs1, src_blk, dst_blk, zeros_f)

    xs2 = pl.pallas_call(
        _tc_mid,
        out_shape=jax.ShapeDtypeStruct((_N, _F), jnp.float32),
    )(z1, xs1, dinv, b1.reshape(1, _F), W2)

    z2 = _prop_kernel(xs2, src_blk, dst_blk, zeros_f)

    out = pl.pallas_call(
        _tc_last,
        out_shape=jax.ShapeDtypeStruct((_N, 40), jnp.float32),
    )(z2, xs2, dinv, b2.reshape(1, _F), Wl, bl.reshape(1, 40))
    return out


# speedup vs baseline: 3.6269x; 1.0522x over previous
import functools

import jax
import jax.numpy as jnp
from jax import lax
from jax.experimental import pallas as pl
from jax.experimental.pallas import tpu as pltpu
from jax.experimental.pallas import tpu_sc as plsc

_N = 10000
_E = 320000
_F = 128
_NC = 2
_NS = 16
_NW = _NC * _NS
_EPW = _E // _NW
_CB = 80
_NCH = _EPW // _CB
_IBC = 25
_NIB = _NCH // _IBC
_NP = 10240
_RPT = _NP // _NS

_mesh = plsc.VectorSubcoreMesh(core_axis_name="c", subcore_axis_name="s")


@functools.partial(
    pl.kernel,
    out_type=jax.ShapeDtypeStruct((_NC, _NP), jnp.float32),
    mesh=_mesh,
    scratch_types=[
        pltpu.VMEM((_NCH, _CB), jnp.int32),
        pltpu.VMEM((_CB,), jnp.float32),
        pltpu.SemaphoreType.DMA,
        pltpu.VMEM_SHARED((_NP,), jnp.float32),
    ],
)
def _deg_kernel(dst_hbm, zeros_hbm, out_hbm, dst_v, ones_v, sem, acc):
    c = lax.axis_index("c")
    s = lax.axis_index("s")
    w = s * _NC + c
    for blk in range(_NIB):
        pltpu.sync_copy(dst_hbm.at[w * _NIB + blk],
                        dst_v.at[pl.ds(blk * _IBC, _IBC)])
    for i in range(_CB // 16):
        ones_v[pl.ds(16 * i, 16)] = jnp.full((16,), 1.0, jnp.float32)
    pltpu.sync_copy(zeros_hbm.at[pl.ds(s * _RPT, _RPT)],
                    acc.at[pl.ds(s * _RPT, _RPT)])
    plsc.subcore_barrier()

    def fire(j, carry):
        pltpu.async_copy(ones_v, acc.at[dst_v.at[j]], sem, add=True)
        return carry

    lax.fori_loop(0, _NCH, fire, 0)

    def drain(j, carry):
        pltpu.make_async_copy(ones_v, acc.at[dst_v.at[0]], sem).wait()
        return carry

    lax.fori_loop(0, _NCH, drain, 0)
    plsc.subcore_barrier()
    pltpu.sync_copy(acc.at[pl.ds(s * _RPT, _RPT)],
                    out_hbm.at[c, pl.ds(s * _RPT, _RPT)])


@functools.partial(
    pl.kernel,
    out_type=jax.ShapeDtypeStruct((_NC, _NP, _F), jnp.float32),
    mesh=_mesh,
    scratch_types=[
        pltpu.VMEM((2, _IBC, _CB), jnp.int32),
        pltpu.VMEM((2, _IBC, _CB), jnp.int32),
        pltpu.VMEM((2, _CB, _F), jnp.float32),
        pltpu.SemaphoreType.DMA,
        pltpu.SemaphoreType.DMA,
        pltpu.SemaphoreType.DMA,
        pltpu.SemaphoreType.DMA,
        pltpu.VMEM_SHARED((_NP, _F), jnp.float32),
    ],
)
def _prop_kernel(xs_hbm, src_hbm, dst_hbm, zeros_hbm, out_hbm,
                 src_v, dst_v, buf, sem0, sem1, zsem, isem, acc):
    c = lax.axis_index("c")
    s = lax.axis_index("s")
    w = s * _NC + c

    sems = (sem0, sem1)

    def load_idx(blk, bank):
        pltpu.async_copy(src_hbm.at[w * _NIB + blk], src_v.at[bank], isem)
        pltpu.async_copy(dst_hbm.at[w * _NIB + blk], dst_v.at[bank], isem)

    def wait_idx(blk, bank):
        pltpu.make_async_copy(src_hbm.at[w * _NIB + blk], src_v.at[bank],
                              isem).wait()
        pltpu.make_async_copy(dst_hbm.at[w * _NIB + blk], dst_v.at[bank],
                              isem).wait()

    def gissue(bank, jj, b):
        pltpu.async_copy(xs_hbm.at[src_v.at[bank, jj]], buf.at[b], sems[b])

    def gwait(bank, jj, b):
        pltpu.make_async_copy(xs_hbm.at[src_v.at[bank, jj]], buf.at[b],
                              sems[b]).wait()

    def scat(bank, jj, b):
        pltpu.sync_copy(buf.at[b], acc.at[dst_v.at[bank, jj]], add=True)

    pltpu.async_copy(zeros_hbm.at[pl.ds(s * _RPT, _RPT)],
                     acc.at[pl.ds(s * _RPT, _RPT)], zsem)
    load_idx(0, 0)
    wait_idx(0, 0)
    gissue(0, 0, 0)
    pltpu.make_async_copy(zeros_hbm.at[pl.ds(s * _RPT, _RPT)],
                          acc.at[pl.ds(s * _RPT, _RPT)], zsem).wait()
    plsc.subcore_barrier()

    for blk in range(_NIB):
        bank = blk % 2
        p = blk % 2
        if blk + 1 < _NIB:
            load_idx(blk + 1, 1 - bank)

        def pair(j2, inner, bank=bank, p=p):
            j = 2 * j2
            gissue(bank, j + 1, 1 - p)
            gwait(bank, j, p)
            scat(bank, j, p)
            gissue(bank, j + 2, p)
            gwait(bank, j + 1, 1 - p)
            scat(bank, j + 1, 1 - p)
            return inner

        lax.fori_loop(0, (_IBC - 1) // 2, pair, 0)
        if blk + 1 < _NIB:
            wait_idx(blk + 1, 1 - bank)
            gissue(1 - bank, 0, 1 - p)
        gwait(bank, _IBC - 1, p)
        scat(bank, _IBC - 1, p)

    plsc.subcore_barrier()
    pltpu.sync_copy(acc.at[pl.ds(s * _RPT, _RPT)],
                    out_hbm.at[c, pl.ds(s * _RPT, _RPT)])


def _tc_first(x_ref, w_ref, deg_ref, xs_ref, dinv_ref):
    dinv = lax.rsqrt(deg_ref[...])
    h = jnp.dot(x_ref[...], w_ref[...], preferred_element_type=jnp.float32)
    dinv_ref[...] = dinv
    xs_ref[...] = dinv * h


def _tc_mid(z_ref, xs_ref, dinv_ref, b_ref, w_ref, out_ref):
    z = z_ref[0, :_N, :] + z_ref[1, :_N, :] + xs_ref[...]
    x1 = jnp.maximum(dinv_ref[...] * z + b_ref[...], 0.0)
    h = jnp.dot(x1, w_ref[...], preferred_element_type=jnp.float32)
    out_ref[...] = dinv_ref[...] * h


def _tc_last(z_ref, xs_ref, dinv_ref, b_ref, wl_ref, bl_ref, out_ref):
    z = z_ref[0, :_N, :] + z_ref[1, :_N, :] + xs_ref[...]
    x2 = jnp.maximum(dinv_ref[...] * z + b_ref[...], 0.0)
    out_ref[...] = (jnp.dot(x2, wl_ref[...], preferred_element_type=jnp.float32)
                    + bl_ref[...])


def kernel(node_features, edge_indices, W1, b1, W2, b2, Wl, bl):
    ei = edge_indices.astype(jnp.int32)
    src_blk = ei[0].reshape(_NW * _NIB, _IBC, _CB)
    dst_blk = ei[1].reshape(_NW * _NIB, _IBC, _CB)
    zeros_f = jnp.zeros((_NP, _F), jnp.float32)
    zeros_1 = jnp.zeros((_NP,), jnp.float32)

    degp = _deg_kernel(dst_blk, zeros_1)
    deg_col = (degp[0, :_N] + degp[1, :_N] + 1.0)[:, None]

    xs1, dinv = pl.pallas_call(
        _tc_first,
        out_shape=[
            jax.ShapeDtypeStruct((_N, _F), jnp.float32),
            jax.ShapeDtypeStruct((_N, 1), jnp.float32),
        ],
    )(node_features, W1, deg_col)

    z1 = _prop_kernel(xs1, src_blk, dst_blk, zeros_f)

    xs2 = pl.pallas_call(
        _tc_mid,
        out_shape=jax.ShapeDtypeStruct((_N, _F), jnp.float32),
    )(z1, xs1, dinv, b1.reshape(1, _F), W2)

    z2 = _prop_kernel(xs2, src_blk, dst_blk, zeros_f)

    out = pl.pallas_call(
        _tc_last,
        out_shape=jax.ShapeDtypeStruct((_N, 40), jnp.float32),
    )(z2, xs2, dinv, b2.reshape(1, _F), Wl, bl.reshape(1, 40))
    return out
